# baseline (speedup 1.0000x reference)
import functools

import jax
import jax.numpy as jnp
from jax.experimental import pallas as pl
from jax.experimental.pallas import tpu as pltpu

_SMOOTHING = 0.1
_CONFIDENCE = 1.0 - _SMOOTHING


def _nll_block(tgt_ref, x_ref, out_ref, acc_ref, *, n_rows, n_cols, br, nblk):
    j = pl.program_id(0)
    x = x_ref[...]
    classes = jax.lax.broadcasted_iota(jnp.int32, (br, n_rows), 0) + j * br
    t = tgt_ref[...]
    eps = _SMOOTHING / (n_cols - 1)
    w = jnp.where(classes == t, -_CONFIDENCE, -eps)
    contrib = jnp.sum(x * w)

    @pl.when(j == 0)
    def _init():
        acc_ref[0] = 0.0

    acc_ref[0] += contrib

    @pl.when(j == nblk - 1)
    def _fin():
        out_ref[0, 0] = acc_ref[0] / n_rows


def kernel(pred, target):
    n_rows, n_cols = pred.shape
    pred_t = pred.T
    br = 1000
    while n_cols % br:
        br //= 2
    nblk = n_cols // br
    tgt2d = target.astype(jnp.int32).reshape(1, n_rows)
    out = pl.pallas_call(
        functools.partial(
            _nll_block, n_rows=n_rows, n_cols=n_cols, br=br, nblk=nblk
        ),
        grid=(nblk,),
        in_specs=[
            pl.BlockSpec((1, n_rows), lambda j: (0, 0)),
            pl.BlockSpec((br, n_rows), lambda j: (j, 0)),
        ],
        out_specs=pl.BlockSpec(
            (1, 1), lambda j: (0, 0), memory_space=pltpu.SMEM
        ),
        out_shape=jax.ShapeDtypeStruct((1, 1), jnp.float32),
        scratch_shapes=[pltpu.SMEM((1,), jnp.float32)],
    )(tgt2d, pred_t)
    return out[0, 0]


# speedup vs baseline: 1.3306x; 1.3306x over previous
import functools

import jax
import jax.numpy as jnp
from jax.experimental import pallas as pl
from jax.experimental.pallas import tpu as pltpu

_SMOOTHING = 0.1
_CONFIDENCE = 1.0 - _SMOOTHING


def _nll_block(tgt_ref, x_ref, out_ref, acc_ref, *, n_rows, n_cols, br, nblk):
    j = pl.program_id(0)
    x = x_ref[...]
    classes = jax.lax.broadcasted_iota(jnp.int32, (br, n_rows), 0) + j * br
    t = tgt_ref[...]
    eps = _SMOOTHING / (n_cols - 1)
    w = jnp.where(classes == t, -_CONFIDENCE, -eps)
    contrib = jnp.sum(x * w)

    @pl.when(j == 0)
    def _init():
        acc_ref[0] = 0.0

    acc_ref[0] += contrib

    @pl.when(j == nblk - 1)
    def _fin():
        out_ref[0, 0] = acc_ref[0] / n_rows


def kernel(pred, target):
    n_rows, n_cols = pred.shape
    pred_t = pred.T
    br = 4000
    while n_cols % br:
        br //= 2
    nblk = n_cols // br
    tgt2d = target.astype(jnp.int32).reshape(1, n_rows)
    out = pl.pallas_call(
        functools.partial(
            _nll_block, n_rows=n_rows, n_cols=n_cols, br=br, nblk=nblk
        ),
        grid=(nblk,),
        in_specs=[
            pl.BlockSpec((1, n_rows), lambda j: (0, 0)),
            pl.BlockSpec((br, n_rows), lambda j: (j, 0)),
        ],
        out_specs=pl.BlockSpec(
            (1, 1), lambda j: (0, 0), memory_space=pltpu.SMEM
        ),
        out_shape=jax.ShapeDtypeStruct((1, 1), jnp.float32),
        scratch_shapes=[pltpu.SMEM((1,), jnp.float32)],
    )(tgt2d, pred_t)
    return out[0, 0]
